# baseline (speedup 1.0000x reference)
import functools

import jax
import jax.numpy as jnp
from jax import lax
from jax.experimental import pallas as pl
from jax.experimental.pallas import tpu as pltpu
from jax.experimental.pallas import tpu_sc as plsc

_CH = 128
_NSUB = 16
_NCORE = 2


@functools.lru_cache(maxsize=None)
def _sc_scatter(n, nchunks, d, with_gather=True):
    nw = _NCORE * _NSUB
    cpt = nchunks // nw
    zr = 200
    nz = n // zr
    nz_rounds = -(-nz // _NSUB)
    mesh = plsc.VectorSubcoreMesh(core_axis_name="c", subcore_axis_name="s")

    scratch = [
        pltpu.VMEM((_CH,), jnp.int32),
        pltpu.VMEM((_CH,), jnp.int32),
        pltpu.VMEM((zr, d), jnp.float32),
        pltpu.VMEM_SHARED((n + _CH, d), jnp.float32),
        pltpu.SemaphoreType.DMA,
        pltpu.SemaphoreType.DMA,
    ]

    def body(y_hbm, src_hbm, dst_hbm, out_hbm, sidx, didx, bufa, acc,
             gsem, ssem):
        c = lax.axis_index("c")
        s = lax.axis_index("s")
        w = c * _NSUB + s

        def fill(buf, val):
            v = jnp.full((16,), val, jnp.float32)

            def row(i, _):
                def col(j, _):
                    buf[i, pl.ds(j * 16, 16)] = v
                    return 0

                return lax.fori_loop(0, d // 16, col, 0)

            lax.fori_loop(0, zr, row, 0)

        fill(bufa, 0.0)
        for j in range(nz_rounds):
            ch = s + j * _NSUB

            @pl.when(ch < nz)
            def _():
                pltpu.sync_copy(bufa.at[pl.ds(0, zr)],
                                acc.at[pl.ds(ch * zr, zr)])

        plsc.subcore_barrier()

        if not with_gather:
            fill(bufa, 1.0)

        ba = bufa.at[pl.ds(0, _CH)]

        def chunk(j, _):
            off = (w * cpt + j) * _CH
            pltpu.sync_copy(dst_hbm.at[pl.ds(off, _CH)], didx)
            if with_gather:
                pltpu.sync_copy(src_hbm.at[pl.ds(off, _CH)], sidx)
                pltpu.async_copy(y_hbm.at[sidx], ba, gsem).wait()
            pltpu.sync_copy(ba, acc.at[didx], add=True)
            return 0

        lax.fori_loop(0, cpt, chunk, 0)

        plsc.subcore_barrier()

        for j in range(nz_rounds):
            ch = s + j * _NSUB

            @pl.when(ch < nz)
            def _():
                pltpu.sync_copy(acc.at[pl.ds(ch * zr, zr)],
                                bufa.at[pl.ds(0, zr)])
                pltpu.sync_copy(bufa.at[pl.ds(0, zr)],
                                out_hbm.at[c, pl.ds(ch * zr, zr)])

    out_type = jax.ShapeDtypeStruct((_NCORE, n, d), jnp.float32)
    if with_gather:
        @functools.partial(pl.kernel, out_type=out_type, mesh=mesh,
                           scratch_types=scratch)
        def kern(y_hbm, src_hbm, dst_hbm, out_hbm, *rest):
            body(y_hbm, src_hbm, dst_hbm, out_hbm, *rest)
    else:
        @functools.partial(pl.kernel, out_type=out_type, mesh=mesh,
                           scratch_types=scratch)
        def kern(dst_hbm, out_hbm, *rest):
            body(None, None, dst_hbm, out_hbm, *rest)

    return kern


@functools.lru_cache(maxsize=None)
def _tc_first(n, d, blk=1000):
    def body(dp_ref, x_ref, w_ref, y_ref, dv_ref):
        deg = dp_ref[0, :, 0:1] + dp_ref[1, :, 0:1] + 1.0
        dinv = lax.rsqrt(deg)
        xw = jnp.dot(x_ref[...], w_ref[...],
                     preferred_element_type=jnp.float32)
        y_ref[...] = xw * dinv
        dv_ref[...] = jnp.broadcast_to(dinv, (blk, 16))

    return pl.pallas_call(
        body,
        grid=(n // blk,),
        in_specs=[
            pl.BlockSpec((2, blk, d), lambda i: (0, i, 0)),
            pl.BlockSpec((blk, d), lambda i: (i, 0)),
            pl.BlockSpec((d, d), lambda i: (0, 0)),
        ],
        out_specs=[
            pl.BlockSpec((blk, d), lambda i: (i, 0)),
            pl.BlockSpec((blk, 16), lambda i: (i, 0)),
        ],
        out_shape=[
            jax.ShapeDtypeStruct((n, d), jnp.float32),
            jax.ShapeDtypeStruct((n, 16), jnp.float32),
        ],
    )


@functools.lru_cache(maxsize=None)
def _tc_mid(n, d, blk=1000):
    def body(dv_ref, acc_ref, y_ref, b_ref, w_ref, out_ref):
        dinv = dv_ref[:, 0:1]
        t = (acc_ref[0] + acc_ref[1] + y_ref[...]) * dinv + b_ref[...][None, :]
        h = jnp.maximum(t, 0.0)
        hw = jnp.dot(h, w_ref[...], preferred_element_type=jnp.float32)
        out_ref[...] = hw * dinv

    return pl.pallas_call(
        body,
        grid=(n // blk,),
        in_specs=[
            pl.BlockSpec((blk, 16), lambda i: (i, 0)),
            pl.BlockSpec((2, blk, d), lambda i: (0, i, 0)),
            pl.BlockSpec((blk, d), lambda i: (i, 0)),
            pl.BlockSpec((d,), lambda i: (0,)),
            pl.BlockSpec((d, d), lambda i: (0, 0)),
        ],
        out_specs=pl.BlockSpec((blk, d), lambda i: (i, 0)),
        out_shape=jax.ShapeDtypeStruct((n, d), jnp.float32),
    )


@functools.lru_cache(maxsize=None)
def _tc_last(n, d, blk=1000):
    def body(dv_ref, acc_ref, y_ref, b_ref, out_ref):
        dinv = dv_ref[:, 0:1]
        out_ref[...] = ((acc_ref[0] + acc_ref[1] + y_ref[...]) * dinv
                        + b_ref[...][None, :])

    return pl.pallas_call(
        body,
        grid=(n // blk,),
        in_specs=[
            pl.BlockSpec((blk, 16), lambda i: (i, 0)),
            pl.BlockSpec((2, blk, d), lambda i: (0, i, 0)),
            pl.BlockSpec((blk, d), lambda i: (i, 0)),
            pl.BlockSpec((d,), lambda i: (0,)),
        ],
        out_specs=pl.BlockSpec((blk, d), lambda i: (i, 0)),
        out_shape=jax.ShapeDtypeStruct((n, d), jnp.float32),
    )


def kernel(x, edge_index, W1, b1, W2, b2, W3, b3):
    n, d = x.shape
    e = edge_index.shape[1]
    assert n % 200 == 0 and d % 16 == 0

    nw = _NCORE * _NSUB
    nchunks = -(-e // (_CH * 2 * nw)) * 2 * nw
    pad = nchunks * _CH - e
    src = jnp.concatenate([edge_index[0], jnp.zeros((pad,), jnp.int32)])
    dst = jnp.concatenate(
        [edge_index[1],
         n + (jnp.arange(pad, dtype=jnp.int32) % _CH)])

    degparts = _sc_scatter(n, nchunks, d, with_gather=False)(dst)
    scatter = _sc_scatter(n, nchunks, d, with_gather=True)

    y1, dinv16 = _tc_first(n, d)(degparts, x, W1)
    acc1 = scatter(y1, src, dst)
    y2 = _tc_mid(n, d)(dinv16, acc1, y1, b1, W2)
    acc2 = scatter(y2, src, dst)
    y3 = _tc_mid(n, d)(dinv16, acc2, y2, b2, W3)
    acc3 = scatter(y3, src, dst)
    return _tc_last(n, d)(dinv16, acc3, y3, b3)


# speedup vs baseline: 2.0622x; 2.0622x over previous
import functools

import jax
import jax.numpy as jnp
from jax import lax
from jax.experimental import pallas as pl
from jax.experimental.pallas import tpu as pltpu
from jax.experimental.pallas import tpu_sc as plsc

_CH = 128
_NSUB = 16
_NCORE = 2


@functools.lru_cache(maxsize=None)
def _sc_scatter(n, nchunks, d, with_gather=True):
    nw = _NCORE * _NSUB
    cpt = nchunks // nw
    zr = 200
    nz = n // zr
    nz_rounds = -(-nz // _NSUB)
    mesh = plsc.VectorSubcoreMesh(core_axis_name="c", subcore_axis_name="s")

    scratch = [
        pltpu.VMEM((_CH,), jnp.int32),
        pltpu.VMEM((_CH,), jnp.int32),
        pltpu.VMEM((zr, d), jnp.float32),
        pltpu.VMEM_SHARED((n + _CH, d), jnp.float32),
        pltpu.SemaphoreType.DMA,
        pltpu.SemaphoreType.DMA,
    ]

    def body(y_hbm, src_hbm, dst_hbm, out_hbm, sidx, didx, bufa, acc,
             gsem, ssem):
        c = lax.axis_index("c")
        s = lax.axis_index("s")
        w = c * _NSUB + s

        def fill(buf, val):
            v = jnp.full((16,), val, jnp.float32)

            def row(i, _):
                def col(j, _):
                    buf[i, pl.ds(j * 16, 16)] = v
                    return 0

                return lax.fori_loop(0, d // 16, col, 0)

            lax.fori_loop(0, zr, row, 0)

        fill(bufa, 0.0)
        for j in range(nz_rounds):
            ch = s + j * _NSUB

            @pl.when(ch < nz)
            def _():
                pltpu.sync_copy(bufa.at[pl.ds(0, zr)],
                                acc.at[pl.ds(ch * zr, zr)])

        plsc.subcore_barrier()

        if not with_gather:
            fill(bufa, 1.0)

        ba = bufa.at[pl.ds(0, _CH)]

        def chunk(j, _):
            off = (w * cpt + j) * _CH
            pltpu.sync_copy(dst_hbm.at[pl.ds(off, _CH)], didx)
            if with_gather:
                pltpu.sync_copy(src_hbm.at[pl.ds(off, _CH)], sidx)
                pltpu.async_copy(y_hbm.at[sidx], ba, gsem).wait()
            pltpu.sync_copy(ba, acc.at[didx], add=True)
            return 0

        lax.fori_loop(0, cpt, chunk, 0)

        plsc.subcore_barrier()

        for j in range(nz_rounds):
            ch = s + j * _NSUB

            @pl.when(ch < nz)
            def _():
                pltpu.sync_copy(acc.at[pl.ds(ch * zr, zr)],
                                bufa.at[pl.ds(0, zr)])
                pltpu.sync_copy(bufa.at[pl.ds(0, zr)],
                                out_hbm.at[c, pl.ds(ch * zr, zr)])

    out_type = jax.ShapeDtypeStruct((_NCORE, n, d), jnp.float32)
    if with_gather:
        @functools.partial(pl.kernel, out_type=out_type, mesh=mesh,
                           scratch_types=scratch)
        def kern(y_hbm, src_hbm, dst_hbm, out_hbm, *rest):
            body(y_hbm, src_hbm, dst_hbm, out_hbm, *rest)
    else:
        @functools.partial(pl.kernel, out_type=out_type, mesh=mesh,
                           scratch_types=scratch)
        def kern(dst_hbm, out_hbm, *rest):
            body(None, None, dst_hbm, out_hbm, *rest)

    return kern


@functools.lru_cache(maxsize=None)
def _tc_first(n, d, blk=1000):
    def body(dp_ref, x_ref, w_ref, y_ref, dv_ref):
        deg = dp_ref[0, :, 0:1] + dp_ref[1, :, 0:1] + 1.0
        dinv = lax.rsqrt(deg)
        xw = jnp.dot(x_ref[...], w_ref[...],
                     preferred_element_type=jnp.float32)
        y_ref[...] = xw * dinv
        dv_ref[...] = jnp.broadcast_to(dinv, (blk, 16))

    return pl.pallas_call(
        body,
        grid=(n // blk,),
        in_specs=[
            pl.BlockSpec((2, blk, d), lambda i: (0, i, 0)),
            pl.BlockSpec((blk, d), lambda i: (i, 0)),
            pl.BlockSpec((d, d), lambda i: (0, 0)),
        ],
        out_specs=[
            pl.BlockSpec((blk, d), lambda i: (i, 0)),
            pl.BlockSpec((blk, 16), lambda i: (i, 0)),
        ],
        out_shape=[
            jax.ShapeDtypeStruct((n, d), jnp.float32),
            jax.ShapeDtypeStruct((n, 16), jnp.float32),
        ],
    )


@functools.lru_cache(maxsize=None)
def _tc_mid(n, d, blk=1000):
    def body(dv_ref, acc_ref, y_ref, b_ref, w_ref, out_ref):
        dinv = dv_ref[:, 0:1]
        t = (acc_ref[0] + acc_ref[1] + y_ref[...]) * dinv + b_ref[...][None, :]
        h = jnp.maximum(t, 0.0)
        hw = jnp.dot(h, w_ref[...], preferred_element_type=jnp.float32)
        out_ref[...] = hw * dinv

    return pl.pallas_call(
        body,
        grid=(n // blk,),
        in_specs=[
            pl.BlockSpec((blk, 16), lambda i: (i, 0)),
            pl.BlockSpec((2, blk, d), lambda i: (0, i, 0)),
            pl.BlockSpec((blk, d), lambda i: (i, 0)),
            pl.BlockSpec((d,), lambda i: (0,)),
            pl.BlockSpec((d, d), lambda i: (0, 0)),
        ],
        out_specs=pl.BlockSpec((blk, d), lambda i: (i, 0)),
        out_shape=jax.ShapeDtypeStruct((n, d), jnp.float32),
    )


@functools.lru_cache(maxsize=None)
def _tc_last(n, d, blk=1000):
    def body(dv_ref, acc_ref, y_ref, b_ref, out_ref):
        dinv = dv_ref[:, 0:1]
        out_ref[...] = ((acc_ref[0] + acc_ref[1] + y_ref[...]) * dinv
                        + b_ref[...][None, :])

    return pl.pallas_call(
        body,
        grid=(n // blk,),
        in_specs=[
            pl.BlockSpec((blk, 16), lambda i: (i, 0)),
            pl.BlockSpec((2, blk, d), lambda i: (0, i, 0)),
            pl.BlockSpec((blk, d), lambda i: (i, 0)),
            pl.BlockSpec((d,), lambda i: (0,)),
        ],
        out_specs=pl.BlockSpec((blk, d), lambda i: (i, 0)),
        out_shape=jax.ShapeDtypeStruct((n, d), jnp.float32),
    )


def kernel(x, edge_index, W1, b1, W2, b2, W3, b3):
    n, d = x.shape
    e = edge_index.shape[1]
    assert n % 200 == 0 and d % 16 == 0

    nw = _NCORE * _NSUB
    nchunks = -(-e // (_CH * 2 * nw)) * 2 * nw
    pad = nchunks * _CH - e
    src = jnp.concatenate(
        [edge_index[0], jnp.arange(pad, dtype=jnp.int32) % n])
    dst = jnp.concatenate(
        [edge_index[1],
         n + (jnp.arange(pad, dtype=jnp.int32) % _CH)])

    degparts = _sc_scatter(n, nchunks, d, with_gather=False)(dst)
    scatter = _sc_scatter(n, nchunks, d, with_gather=True)

    y1, dinv16 = _tc_first(n, d)(degparts, x, W1)
    acc1 = scatter(y1, src, dst)
    y2 = _tc_mid(n, d)(dinv16, acc1, y1, b1, W2)
    acc2 = scatter(y2, src, dst)
    y3 = _tc_mid(n, d)(dinv16, acc2, y2, b2, W3)
    acc3 = scatter(y3, src, dst)
    return _tc_last(n, d)(dinv16, acc3, y3, b3)


# speedup vs baseline: 2.4003x; 1.1640x over previous
import functools

import jax
import jax.numpy as jnp
from jax import lax
from jax.experimental import pallas as pl
from jax.experimental.pallas import tpu as pltpu
from jax.experimental.pallas import tpu_sc as plsc

_CH = 128
_NSUB = 16
_NCORE = 2


@functools.lru_cache(maxsize=None)
def _sc_scatter(n, nchunks, d, with_gather=True):
    nw = _NCORE * _NSUB
    cpt = nchunks // nw
    zr = 40
    nz = n // zr
    nz_rounds = -(-nz // _NSUB)
    grp = 8
    assert cpt % grp == 0
    mesh = plsc.VectorSubcoreMesh(core_axis_name="c", subcore_axis_name="s")

    scratch = [
        (pltpu.VMEM((_CH,), jnp.int32),
         pltpu.VMEM((_CH,), jnp.int32)),
        (pltpu.VMEM((_CH,), jnp.int32),
         pltpu.VMEM((_CH,), jnp.int32)),
        (pltpu.VMEM((_CH, d), jnp.float32),
         pltpu.VMEM((_CH, d), jnp.float32)),
        pltpu.VMEM((zr, d), jnp.float32),
        pltpu.VMEM_SHARED((n + _CH, d), jnp.float32),
        (pltpu.SemaphoreType.DMA, pltpu.SemaphoreType.DMA),
        (pltpu.SemaphoreType.DMA, pltpu.SemaphoreType.DMA),
    ]

    def body(y_hbm, src_hbm, dst_hbm, out_hbm, sidx, didx, buf, zbuf, acc,
             gsem, ssem):
        c = lax.axis_index("c")
        s = lax.axis_index("s")
        w = c * _NSUB + s

        def fill(b, nrows, val):
            v = jnp.full((16,), val, jnp.float32)

            def row(i, _):
                def col(j, _):
                    b[i, pl.ds(j * 16, 16)] = v
                    return 0

                return lax.fori_loop(0, d // 16, col, 0)

            lax.fori_loop(0, nrows, row, 0)

        fill(zbuf, zr, 0.0)
        for j in range(nz_rounds):
            ch = s + j * _NSUB

            @pl.when(ch < nz)
            def _():
                pltpu.sync_copy(zbuf, acc.at[pl.ds(ch * zr, zr)])

        plsc.subcore_barrier()

        if with_gather:
            def group(g, _):
                descs = [None] * grp
                for k in range(grp):
                    p = k % 2
                    if k >= 2:
                        descs[k - 2].wait()
                    off = (w * cpt + g * grp + k) * _CH
                    pltpu.sync_copy(src_hbm.at[pl.ds(off, _CH)], sidx[p])
                    pltpu.sync_copy(dst_hbm.at[pl.ds(off, _CH)], didx[p])
                    pltpu.async_copy(y_hbm.at[sidx[p]], buf[p],
                                     gsem[p]).wait()
                    descs[k] = pltpu.async_copy(buf[p], acc.at[didx[p]],
                                                ssem[p], add=True)
                descs[grp - 2].wait()
                descs[grp - 1].wait()
                return 0

            lax.fori_loop(0, cpt // grp, group, 0)
        else:
            fill(buf[0], _CH, 1.0)

            def chunk(j, _):
                off = (w * cpt + j) * _CH
                pltpu.sync_copy(dst_hbm.at[pl.ds(off, _CH)], didx[0])
                pltpu.sync_copy(buf[0], acc.at[didx[0]], add=True)
                return 0

            lax.fori_loop(0, cpt, chunk, 0)

        plsc.subcore_barrier()

        for j in range(nz_rounds):
            ch = s + j * _NSUB

            @pl.when(ch < nz)
            def _():
                pltpu.sync_copy(acc.at[pl.ds(ch * zr, zr)], zbuf)
                pltpu.sync_copy(zbuf, out_hbm.at[c, pl.ds(ch * zr, zr)])

    out_type = jax.ShapeDtypeStruct((_NCORE, n, d), jnp.float32)
    if with_gather:
        @functools.partial(pl.kernel, out_type=out_type, mesh=mesh,
                           scratch_types=scratch)
        def kern(y_hbm, src_hbm, dst_hbm, out_hbm, *rest):
            body(y_hbm, src_hbm, dst_hbm, out_hbm, *rest)
    else:
        @functools.partial(pl.kernel, out_type=out_type, mesh=mesh,
                           scratch_types=scratch)
        def kern(dst_hbm, out_hbm, *rest):
            body(None, None, dst_hbm, out_hbm, *rest)

    return kern


@functools.lru_cache(maxsize=None)
def _tc_first(n, d, blk=1000):
    def body(dp_ref, x_ref, w_ref, y_ref, dv_ref):
        deg = dp_ref[0, :, 0:1] + dp_ref[1, :, 0:1] + 1.0
        dinv = lax.rsqrt(deg)
        xw = jnp.dot(x_ref[...], w_ref[...],
                     preferred_element_type=jnp.float32)
        y_ref[...] = xw * dinv
        dv_ref[...] = jnp.broadcast_to(dinv, (blk, 16))

    return pl.pallas_call(
        body,
        grid=(n // blk,),
        in_specs=[
            pl.BlockSpec((2, blk, d), lambda i: (0, i, 0)),
            pl.BlockSpec((blk, d), lambda i: (i, 0)),
            pl.BlockSpec((d, d), lambda i: (0, 0)),
        ],
        out_specs=[
            pl.BlockSpec((blk, d), lambda i: (i, 0)),
            pl.BlockSpec((blk, 16), lambda i: (i, 0)),
        ],
        out_shape=[
            jax.ShapeDtypeStruct((n, d), jnp.float32),
            jax.ShapeDtypeStruct((n, 16), jnp.float32),
        ],
    )


@functools.lru_cache(maxsize=None)
def _tc_mid(n, d, blk=1000):
    def body(dv_ref, acc_ref, y_ref, b_ref, w_ref, out_ref):
        dinv = dv_ref[:, 0:1]
        t = (acc_ref[0] + acc_ref[1] + y_ref[...]) * dinv + b_ref[...][None, :]
        h = jnp.maximum(t, 0.0)
        hw = jnp.dot(h, w_ref[...], preferred_element_type=jnp.float32)
        out_ref[...] = hw * dinv

    return pl.pallas_call(
        body,
        grid=(n // blk,),
        in_specs=[
            pl.BlockSpec((blk, 16), lambda i: (i, 0)),
            pl.BlockSpec((2, blk, d), lambda i: (0, i, 0)),
            pl.BlockSpec((blk, d), lambda i: (i, 0)),
            pl.BlockSpec((d,), lambda i: (0,)),
            pl.BlockSpec((d, d), lambda i: (0, 0)),
        ],
        out_specs=pl.BlockSpec((blk, d), lambda i: (i, 0)),
        out_shape=jax.ShapeDtypeStruct((n, d), jnp.float32),
    )


@functools.lru_cache(maxsize=None)
def _tc_last(n, d, blk=1000):
    def body(dv_ref, acc_ref, y_ref, b_ref, out_ref):
        dinv = dv_ref[:, 0:1]
        out_ref[...] = ((acc_ref[0] + acc_ref[1] + y_ref[...]) * dinv
                        + b_ref[...][None, :])

    return pl.pallas_call(
        body,
        grid=(n // blk,),
        in_specs=[
            pl.BlockSpec((blk, 16), lambda i: (i, 0)),
            pl.BlockSpec((2, blk, d), lambda i: (0, i, 0)),
            pl.BlockSpec((blk, d), lambda i: (i, 0)),
            pl.BlockSpec((d,), lambda i: (0,)),
        ],
        out_specs=pl.BlockSpec((blk, d), lambda i: (i, 0)),
        out_shape=jax.ShapeDtypeStruct((n, d), jnp.float32),
    )


def kernel(x, edge_index, W1, b1, W2, b2, W3, b3):
    n, d = x.shape
    e = edge_index.shape[1]
    assert n % 200 == 0 and d % 16 == 0

    nw = _NCORE * _NSUB
    nchunks = -(-e // (_CH * 2 * nw)) * 2 * nw
    pad = nchunks * _CH - e
    src = jnp.concatenate(
        [edge_index[0], jnp.arange(pad, dtype=jnp.int32) % n])
    dst = jnp.concatenate(
        [edge_index[1],
         n + (jnp.arange(pad, dtype=jnp.int32) % _CH)])

    degparts = _sc_scatter(n, nchunks, d, with_gather=False)(dst)
    scatter = _sc_scatter(n, nchunks, d, with_gather=True)

    y1, dinv16 = _tc_first(n, d)(degparts, x, W1)
    acc1 = scatter(y1, src, dst)
    y2 = _tc_mid(n, d)(dinv16, acc1, y1, b1, W2)
    acc2 = scatter(y2, src, dst)
    y3 = _tc_mid(n, d)(dinv16, acc2, y2, b2, W3)
    acc3 = scatter(y3, src, dst)
    return _tc_last(n, d)(dinv16, acc3, y3, b3)


# speedup vs baseline: 2.4862x; 1.0358x over previous
import functools

import jax
import jax.numpy as jnp
from jax import lax
from jax.experimental import pallas as pl
from jax.experimental.pallas import tpu as pltpu
from jax.experimental.pallas import tpu_sc as plsc

_CH = 128
_NSUB = 16
_NCORE = 2


@functools.lru_cache(maxsize=None)
def _sc_scatter(n, nchunks, d, with_gather=True):
    nw = _NCORE * _NSUB
    cpt = nchunks // nw
    zr = 40
    nz = n // zr
    nz_rounds = -(-nz // _NSUB)
    grp = 8
    assert cpt % grp == 0
    mesh = plsc.VectorSubcoreMesh(core_axis_name="c", subcore_axis_name="s")

    scratch = [
        (pltpu.VMEM((_CH,), jnp.int32),
         pltpu.VMEM((_CH,), jnp.int32)),
        (pltpu.VMEM((_CH,), jnp.int32),
         pltpu.VMEM((_CH,), jnp.int32)),
        (pltpu.VMEM((_CH, d), jnp.float32),
         pltpu.VMEM((_CH, d), jnp.float32)),
        pltpu.VMEM((zr, d), jnp.float32),
        pltpu.VMEM_SHARED((n + _CH, d), jnp.float32),
        (pltpu.SemaphoreType.DMA, pltpu.SemaphoreType.DMA),
        (pltpu.SemaphoreType.DMA, pltpu.SemaphoreType.DMA),
    ]

    def body(y_hbm, src_hbm, dst_hbm, out_hbm, sidx, didx, buf, zbuf, acc,
             gsem, ssem):
        c = lax.axis_index("c")
        s = lax.axis_index("s")
        w = c * _NSUB + s

        def fill(b, nrows, val):
            v = jnp.full((16,), val, jnp.float32)

            def row(i, _):
                def col(j, _):
                    b[i, pl.ds(j * 16, 16)] = v
                    return 0

                return lax.fori_loop(0, d // 16, col, 0)

            lax.fori_loop(0, nrows, row, 0)

        fill(zbuf, zr, 0.0)
        for j in range(nz_rounds):
            ch = s + j * _NSUB

            @pl.when(ch < nz)
            def _():
                pltpu.sync_copy(zbuf, acc.at[pl.ds(ch * zr, zr)])

        plsc.subcore_barrier()

        if with_gather:
            def group(g, _):
                descs = [None] * grp
                for k in range(grp):
                    p = k % 2
                    if k >= 2:
                        descs[k - 2].wait()
                    off = (w * cpt + g * grp + k) * _CH
                    pltpu.sync_copy(src_hbm.at[pl.ds(off, _CH)], sidx[p])
                    pltpu.sync_copy(dst_hbm.at[pl.ds(off, _CH)], didx[p])
                    pltpu.async_copy(y_hbm.at[sidx[p]], buf[p],
                                     gsem[p]).wait()
                    descs[k] = pltpu.async_copy(buf[p], acc.at[didx[p]],
                                                ssem[p], add=True)
                descs[grp - 2].wait()
                descs[grp - 1].wait()
                return 0

            lax.fori_loop(0, cpt // grp, group, 0)
        else:
            fill(buf[0], _CH, 1.0)
            fill(buf[1], _CH, 1.0)

            def group(g, _):
                descs = [None] * grp
                for k in range(grp):
                    p = k % 2
                    if k >= 2:
                        descs[k - 2].wait()
                    off = (w * cpt + g * grp + k) * _CH
                    pltpu.sync_copy(dst_hbm.at[pl.ds(off, _CH)], didx[p])
                    descs[k] = pltpu.async_copy(buf[p], acc.at[didx[p]],
                                                ssem[p], add=True)
                descs[grp - 2].wait()
                descs[grp - 1].wait()
                return 0

            lax.fori_loop(0, cpt // grp, group, 0)

        plsc.subcore_barrier()

        for j in range(nz_rounds):
            ch = s + j * _NSUB

            @pl.when(ch < nz)
            def _():
                pltpu.sync_copy(acc.at[pl.ds(ch * zr, zr)], zbuf)
                pltpu.sync_copy(zbuf, out_hbm.at[c, pl.ds(ch * zr, zr)])

    out_type = jax.ShapeDtypeStruct((_NCORE, n, d), jnp.float32)
    if with_gather:
        @functools.partial(pl.kernel, out_type=out_type, mesh=mesh,
                           scratch_types=scratch)
        def kern(y_hbm, src_hbm, dst_hbm, out_hbm, *rest):
            body(y_hbm, src_hbm, dst_hbm, out_hbm, *rest)
    else:
        @functools.partial(pl.kernel, out_type=out_type, mesh=mesh,
                           scratch_types=scratch)
        def kern(dst_hbm, out_hbm, *rest):
            body(None, None, dst_hbm, out_hbm, *rest)

    return kern


@functools.lru_cache(maxsize=None)
def _tc_first(n, d, blk=1000):
    def body(dp_ref, x_ref, w_ref, y_ref, dv_ref):
        deg = dp_ref[0, :, 0:1] + dp_ref[1, :, 0:1] + 1.0
        dinv = lax.rsqrt(deg)
        xw = jnp.dot(x_ref[...], w_ref[...],
                     preferred_element_type=jnp.float32)
        y_ref[...] = xw * dinv
        dv_ref[...] = jnp.broadcast_to(dinv, (blk, 16))

    return pl.pallas_call(
        body,
        grid=(n // blk,),
        in_specs=[
            pl.BlockSpec((2, blk, d), lambda i: (0, i, 0)),
            pl.BlockSpec((blk, d), lambda i: (i, 0)),
            pl.BlockSpec((d, d), lambda i: (0, 0)),
        ],
        out_specs=[
            pl.BlockSpec((blk, d), lambda i: (i, 0)),
            pl.BlockSpec((blk, 16), lambda i: (i, 0)),
        ],
        out_shape=[
            jax.ShapeDtypeStruct((n, d), jnp.float32),
            jax.ShapeDtypeStruct((n, 16), jnp.float32),
        ],
    )


@functools.lru_cache(maxsize=None)
def _tc_mid(n, d, blk=1000):
    def body(dv_ref, acc_ref, y_ref, b_ref, w_ref, out_ref):
        dinv = dv_ref[:, 0:1]
        t = (acc_ref[0] + acc_ref[1] + y_ref[...]) * dinv + b_ref[...][None, :]
        h = jnp.maximum(t, 0.0)
        hw = jnp.dot(h, w_ref[...], preferred_element_type=jnp.float32)
        out_ref[...] = hw * dinv

    return pl.pallas_call(
        body,
        grid=(n // blk,),
        in_specs=[
            pl.BlockSpec((blk, 16), lambda i: (i, 0)),
            pl.BlockSpec((2, blk, d), lambda i: (0, i, 0)),
            pl.BlockSpec((blk, d), lambda i: (i, 0)),
            pl.BlockSpec((d,), lambda i: (0,)),
            pl.BlockSpec((d, d), lambda i: (0, 0)),
        ],
        out_specs=pl.BlockSpec((blk, d), lambda i: (i, 0)),
        out_shape=jax.ShapeDtypeStruct((n, d), jnp.float32),
    )


@functools.lru_cache(maxsize=None)
def _tc_last(n, d, blk=1000):
    def body(dv_ref, acc_ref, y_ref, b_ref, out_ref):
        dinv = dv_ref[:, 0:1]
        out_ref[...] = ((acc_ref[0] + acc_ref[1] + y_ref[...]) * dinv
                        + b_ref[...][None, :])

    return pl.pallas_call(
        body,
        grid=(n // blk,),
        in_specs=[
            pl.BlockSpec((blk, 16), lambda i: (i, 0)),
            pl.BlockSpec((2, blk, d), lambda i: (0, i, 0)),
            pl.BlockSpec((blk, d), lambda i: (i, 0)),
            pl.BlockSpec((d,), lambda i: (0,)),
        ],
        out_specs=pl.BlockSpec((blk, d), lambda i: (i, 0)),
        out_shape=jax.ShapeDtypeStruct((n, d), jnp.float32),
    )


def kernel(x, edge_index, W1, b1, W2, b2, W3, b3):
    n, d = x.shape
    e = edge_index.shape[1]
    assert n % 200 == 0 and d % 16 == 0

    nw = _NCORE * _NSUB
    nchunks = -(-e // (_CH * 2 * nw)) * 2 * nw
    pad = nchunks * _CH - e
    src = jnp.concatenate(
        [edge_index[0], jnp.arange(pad, dtype=jnp.int32) % n])
    dst = jnp.concatenate(
        [edge_index[1],
         n + (jnp.arange(pad, dtype=jnp.int32) % _CH)])

    degparts = _sc_scatter(n, nchunks, d, with_gather=False)(dst)
    scatter = _sc_scatter(n, nchunks, d, with_gather=True)

    y1, dinv16 = _tc_first(n, d)(degparts, x, W1)
    acc1 = scatter(y1, src, dst)
    y2 = _tc_mid(n, d)(dinv16, acc1, y1, b1, W2)
    acc2 = scatter(y2, src, dst)
    y3 = _tc_mid(n, d)(dinv16, acc2, y2, b2, W3)
    acc3 = scatter(y3, src, dst)
    return _tc_last(n, d)(dinv16, acc3, y3, b3)


# speedup vs baseline: 2.5235x; 1.0150x over previous
import functools

import jax
import jax.numpy as jnp
from jax import lax
from jax.experimental import pallas as pl
from jax.experimental.pallas import tpu as pltpu
from jax.experimental.pallas import tpu_sc as plsc

_CH = 128
_NSUB = 16
_NCORE = 2


@functools.lru_cache(maxsize=None)
def _sc_scatter(n, nchunks, d, with_gather=True):
    nw = _NCORE * _NSUB
    cpt = nchunks // nw
    zr = 40
    nz = n // zr
    nz_rounds = -(-nz // _NSUB)
    grp = 16
    assert cpt % grp == 0
    mesh = plsc.VectorSubcoreMesh(core_axis_name="c", subcore_axis_name="s")

    scratch = [
        (pltpu.VMEM((_CH,), jnp.int32),
         pltpu.VMEM((_CH,), jnp.int32)),
        (pltpu.VMEM((_CH,), jnp.int32),
         pltpu.VMEM((_CH,), jnp.int32)),
        (pltpu.VMEM((_CH, d), jnp.float32),
         pltpu.VMEM((_CH, d), jnp.float32)),
        pltpu.VMEM((zr, d), jnp.float32),
        pltpu.VMEM_SHARED((n + _CH, d), jnp.float32),
        (pltpu.SemaphoreType.DMA, pltpu.SemaphoreType.DMA),
        (pltpu.SemaphoreType.DMA, pltpu.SemaphoreType.DMA),
    ]

    def body(y_hbm, src_hbm, dst_hbm, out_hbm, sidx, didx, buf, zbuf, acc,
             gsem, ssem):
        c = lax.axis_index("c")
        s = lax.axis_index("s")
        w = c * _NSUB + s

        def fill(b, nrows, val):
            v = jnp.full((16,), val, jnp.float32)

            def row(i, _):
                def col(j, _):
                    b[i, pl.ds(j * 16, 16)] = v
                    return 0

                return lax.fori_loop(0, d // 16, col, 0)

            lax.fori_loop(0, nrows, row, 0)

        fill(zbuf, zr, 0.0)
        for j in range(nz_rounds):
            ch = s + j * _NSUB

            @pl.when(ch < nz)
            def _():
                pltpu.sync_copy(zbuf, acc.at[pl.ds(ch * zr, zr)])

        plsc.subcore_barrier()

        if with_gather:
            def group(g, _):
                descs = [None] * grp
                for k in range(grp):
                    p = k % 2
                    if k >= 2:
                        descs[k - 2].wait()
                    off = (w * cpt + g * grp + k) * _CH
                    pltpu.sync_copy(src_hbm.at[pl.ds(off, _CH)], sidx[p])
                    pltpu.sync_copy(dst_hbm.at[pl.ds(off, _CH)], didx[p])
                    pltpu.async_copy(y_hbm.at[sidx[p]], buf[p],
                                     gsem[p]).wait()
                    descs[k] = pltpu.async_copy(buf[p], acc.at[didx[p]],
                                                ssem[p], add=True)
                descs[grp - 2].wait()
                descs[grp - 1].wait()
                return 0

            lax.fori_loop(0, cpt // grp, group, 0)
        else:
            fill(buf[0], _CH, 1.0)
            fill(buf[1], _CH, 1.0)

            def group(g, _):
                descs = [None] * grp
                for k in range(grp):
                    p = k % 2
                    if k >= 2:
                        descs[k - 2].wait()
                    off = (w * cpt + g * grp + k) * _CH
                    pltpu.sync_copy(dst_hbm.at[pl.ds(off, _CH)], didx[p])
                    descs[k] = pltpu.async_copy(buf[p], acc.at[didx[p]],
                                                ssem[p], add=True)
                descs[grp - 2].wait()
                descs[grp - 1].wait()
                return 0

            lax.fori_loop(0, cpt // grp, group, 0)

        plsc.subcore_barrier()

        for j in range(nz_rounds):
            ch = s + j * _NSUB

            @pl.when(ch < nz)
            def _():
                pltpu.sync_copy(acc.at[pl.ds(ch * zr, zr)], zbuf)
                pltpu.sync_copy(zbuf, out_hbm.at[c, pl.ds(ch * zr, zr)])

    out_type = jax.ShapeDtypeStruct((_NCORE, n, d), jnp.float32)
    if with_gather:
        @functools.partial(pl.kernel, out_type=out_type, mesh=mesh,
                           scratch_types=scratch)
        def kern(y_hbm, src_hbm, dst_hbm, out_hbm, *rest):
            body(y_hbm, src_hbm, dst_hbm, out_hbm, *rest)
    else:
        @functools.partial(pl.kernel, out_type=out_type, mesh=mesh,
                           scratch_types=scratch)
        def kern(dst_hbm, out_hbm, *rest):
            body(None, None, dst_hbm, out_hbm, *rest)

    return kern


@functools.lru_cache(maxsize=None)
def _tc_first(n, d, blk=1000):
    def body(dp_ref, x_ref, w_ref, y_ref, dv_ref):
        deg = dp_ref[0, :, 0:1] + dp_ref[1, :, 0:1] + 1.0
        dinv = lax.rsqrt(deg)
        xw = jnp.dot(x_ref[...], w_ref[...],
                     preferred_element_type=jnp.float32)
        y_ref[...] = xw * dinv
        dv_ref[...] = jnp.broadcast_to(dinv, (blk, 16))

    return pl.pallas_call(
        body,
        grid=(n // blk,),
        in_specs=[
            pl.BlockSpec((2, blk, d), lambda i: (0, i, 0)),
            pl.BlockSpec((blk, d), lambda i: (i, 0)),
            pl.BlockSpec((d, d), lambda i: (0, 0)),
        ],
        out_specs=[
            pl.BlockSpec((blk, d), lambda i: (i, 0)),
            pl.BlockSpec((blk, 16), lambda i: (i, 0)),
        ],
        out_shape=[
            jax.ShapeDtypeStruct((n, d), jnp.float32),
            jax.ShapeDtypeStruct((n, 16), jnp.float32),
        ],
    )


@functools.lru_cache(maxsize=None)
def _tc_mid(n, d, blk=1000):
    def body(dv_ref, acc_ref, y_ref, b_ref, w_ref, out_ref):
        dinv = dv_ref[:, 0:1]
        t = (acc_ref[0] + acc_ref[1] + y_ref[...]) * dinv + b_ref[...][None, :]
        h = jnp.maximum(t, 0.0)
        hw = jnp.dot(h, w_ref[...], preferred_element_type=jnp.float32)
        out_ref[...] = hw * dinv

    return pl.pallas_call(
        body,
        grid=(n // blk,),
        in_specs=[
            pl.BlockSpec((blk, 16), lambda i: (i, 0)),
            pl.BlockSpec((2, blk, d), lambda i: (0, i, 0)),
            pl.BlockSpec((blk, d), lambda i: (i, 0)),
            pl.BlockSpec((d,), lambda i: (0,)),
            pl.BlockSpec((d, d), lambda i: (0, 0)),
        ],
        out_specs=pl.BlockSpec((blk, d), lambda i: (i, 0)),
        out_shape=jax.ShapeDtypeStruct((n, d), jnp.float32),
    )


@functools.lru_cache(maxsize=None)
def _tc_last(n, d, blk=1000):
    def body(dv_ref, acc_ref, y_ref, b_ref, out_ref):
        dinv = dv_ref[:, 0:1]
        out_ref[...] = ((acc_ref[0] + acc_ref[1] + y_ref[...]) * dinv
                        + b_ref[...][None, :])

    return pl.pallas_call(
        body,
        grid=(n // blk,),
        in_specs=[
            pl.BlockSpec((blk, 16), lambda i: (i, 0)),
            pl.BlockSpec((2, blk, d), lambda i: (0, i, 0)),
            pl.BlockSpec((blk, d), lambda i: (i, 0)),
            pl.BlockSpec((d,), lambda i: (0,)),
        ],
        out_specs=pl.BlockSpec((blk, d), lambda i: (i, 0)),
        out_shape=jax.ShapeDtypeStruct((n, d), jnp.float32),
    )


def kernel(x, edge_index, W1, b1, W2, b2, W3, b3):
    n, d = x.shape
    e = edge_index.shape[1]
    assert n % 200 == 0 and d % 16 == 0

    nw = _NCORE * _NSUB
    nchunks = -(-e // (_CH * 2 * nw)) * 2 * nw
    pad = nchunks * _CH - e
    src = jnp.concatenate(
        [edge_index[0], jnp.arange(pad, dtype=jnp.int32) % n])
    dst = jnp.concatenate(
        [edge_index[1],
         n + (jnp.arange(pad, dtype=jnp.int32) % _CH)])

    degparts = _sc_scatter(n, nchunks, d, with_gather=False)(dst)
    scatter = _sc_scatter(n, nchunks, d, with_gather=True)

    y1, dinv16 = _tc_first(n, d)(degparts, x, W1)
    acc1 = scatter(y1, src, dst)
    y2 = _tc_mid(n, d)(dinv16, acc1, y1, b1, W2)
    acc2 = scatter(y2, src, dst)
    y3 = _tc_mid(n, d)(dinv16, acc2, y2, b2, W3)
    acc3 = scatter(y3, src, dst)
    return _tc_last(n, d)(dinv16, acc3, y3, b3)
